# baseline (speedup 1.0000x reference)
import functools

import jax
import jax.numpy as jnp
from jax import lax
from jax.experimental import pallas as pl
from jax.experimental.pallas import tpu as pltpu, tpu_sc as plsc

ROLE_CAD, ROLE_EMB = 13, 8
RACE_CAD, RACE_EMB = 5, 4
GEND_CAD, GEND_EMB = 3, 2
ALIGN_CAD, ALIGN_EMB = 3, 2
B = 16384
D = ROLE_EMB + RACE_EMB + GEND_EMB + ALIGN_EMB

NC, NS, L = 2, 16, 16
NW = NC * NS
BPW = B // NW
CHUNKS = BPW // L

OFF_ROLE = 0
OFF_RACE = OFF_ROLE + ROLE_CAD * ROLE_EMB
OFF_GEND = OFF_RACE + RACE_CAD * RACE_EMB
OFF_ALIGN = OFF_GEND + GEND_CAD * GEND_EMB
FLAT_LEN = OFF_ALIGN + ALIGN_CAD * ALIGN_EMB


def _hero_body(role_h, race_h, gend_h, align_h, rt_h, ct_h, gt_h, at_h,
               out_h, rt_v, ct_v, gt_v, at_v, ri_v, ci_v, gi_v, ai_v, out_v):
    wid = lax.axis_index("s") * NC + lax.axis_index("c")
    base = wid * BPW

    pltpu.sync_copy(rt_h, rt_v)
    pltpu.sync_copy(ct_h, ct_v)
    pltpu.sync_copy(gt_h, gt_v)
    pltpu.sync_copy(at_h, at_v)
    pltpu.sync_copy(role_h.at[pl.ds(base, BPW)], ri_v)
    pltpu.sync_copy(race_h.at[pl.ds(base, BPW)], ci_v)
    pltpu.sync_copy(gend_h.at[pl.ds(base, BPW)], gi_v)
    pltpu.sync_copy(align_h.at[pl.ds(base, BPW)], ai_v)

    lane = lax.iota(jnp.int32, L)
    col_vecs = [jnp.full((L,), col, jnp.int32) for col in range(ROLE_EMB)]

    @plsc.parallel_loop(0, CHUNKS, step=1, unroll=4)
    def chunk_body(k):
        e0 = k * L
        r = ri_v[pl.ds(e0, L)]
        c = ci_v[pl.ds(e0, L)]
        g = gi_v[pl.ds(e0, L)]
        a = ai_v[pl.ds(e0, L)]
        a = jnp.minimum(jnp.maximum(a + 1, 0), ALIGN_CAD - 1)
        for col in range(D):
            if col < ROLE_EMB:
                vals = plsc.load_gather(rt_v, [r, col_vecs[col]])
            elif col < ROLE_EMB + RACE_EMB:
                vals = plsc.load_gather(ct_v, [c, col_vecs[col - ROLE_EMB]])
            elif col < ROLE_EMB + RACE_EMB + GEND_EMB:
                vals = plsc.load_gather(
                    gt_v, [g, col_vecs[col - ROLE_EMB - RACE_EMB]])
            else:
                vals = plsc.load_gather(
                    at_v,
                    [a, col_vecs[col - ROLE_EMB - RACE_EMB - GEND_EMB]])
            out_v[col, pl.ds(e0, L)] = vals

    pltpu.sync_copy(out_v, out_h.at[:, pl.ds(base, BPW)])


_hero = functools.partial(
    pl.kernel,
    out_type=jax.ShapeDtypeStruct((D, B), jnp.float32),
    mesh=plsc.VectorSubcoreMesh(core_axis_name="c", subcore_axis_name="s"),
    compiler_params=pltpu.CompilerParams(needs_layout_passes=False),
    scratch_types=[
        pltpu.VMEM((ROLE_CAD, ROLE_EMB), jnp.float32),
        pltpu.VMEM((RACE_CAD, RACE_EMB), jnp.float32),
        pltpu.VMEM((GEND_CAD, GEND_EMB), jnp.float32),
        pltpu.VMEM((ALIGN_CAD, ALIGN_EMB), jnp.float32),
        pltpu.VMEM((BPW,), jnp.int32),
        pltpu.VMEM((BPW,), jnp.int32),
        pltpu.VMEM((BPW,), jnp.int32),
        pltpu.VMEM((BPW,), jnp.int32),
        pltpu.VMEM((D, BPW), jnp.float32),
    ],
)(_hero_body)


def kernel(role, race, gend, align, role_table, race_table, gend_table,
           align_table):
    out_cm = _hero(role.astype(jnp.int32), race.astype(jnp.int32),
                   gend.astype(jnp.int32), align.astype(jnp.int32),
                   role_table, race_table, gend_table, align_table)
    return out_cm.T


# speedup vs baseline: 1.0689x; 1.0689x over previous
import functools

import jax
import jax.numpy as jnp
from jax import lax
from jax.experimental import pallas as pl
from jax.experimental.pallas import tpu as pltpu, tpu_sc as plsc

ROLE_CAD, ROLE_EMB = 13, 8
RACE_CAD, RACE_EMB = 5, 4
GEND_CAD, GEND_EMB = 3, 2
ALIGN_CAD, ALIGN_EMB = 3, 2
B = 16384
D = ROLE_EMB + RACE_EMB + GEND_EMB + ALIGN_EMB

NC, NS, L = 2, 16, 16
NW = NC * NS
BPW = B // NW
CHUNKS = BPW // L

TAB_ROWS, TAB_COLS = 16, 128
FLAT_LEN = TAB_ROWS * TAB_COLS


def _hero_body(role_h, race_h, gend_h, align_h, tab_h, out_h,
               flat_v, ri_v, ci_v, gi_v, ai_v, out_v):
    wid = lax.axis_index("s") * NC + lax.axis_index("c")
    base = wid * BPW

    pltpu.sync_copy(tab_h, flat_v)
    pltpu.sync_copy(role_h.at[pl.ds(base, BPW)], ri_v)
    pltpu.sync_copy(race_h.at[pl.ds(base, BPW)], ci_v)
    pltpu.sync_copy(gend_h.at[pl.ds(base, BPW)], gi_v)
    pltpu.sync_copy(align_h.at[pl.ds(base, BPW)], ai_v)

    lane = lax.iota(jnp.int32, L)

    @plsc.parallel_loop(0, CHUNKS, step=1, unroll=4)
    def chunk_body(k):
        e0 = k * L
        r = ri_v[pl.ds(e0, L)]
        c = ci_v[pl.ds(e0, L)]
        g = gi_v[pl.ds(e0, L)]
        a = ai_v[pl.ds(e0, L)]
        a = jnp.minimum(jnp.maximum(a + 1, 0), ALIGN_CAD - 1)
        r = r * TAB_COLS
        c = c * TAB_COLS
        g = g * TAB_COLS
        a = a * TAB_COLS
        for col in range(D):
            if col < ROLE_EMB:
                idx = r
            elif col < ROLE_EMB + RACE_EMB:
                idx = c
            elif col < ROLE_EMB + RACE_EMB + GEND_EMB:
                idx = g
            else:
                idx = a
            vals = plsc.load_gather(flat_v, [idx + col])
            out_v[col, pl.ds(e0, L)] = vals

    pltpu.sync_copy(out_v, out_h.at[:, pl.ds(base, BPW)])


_hero = functools.partial(
    pl.kernel,
    out_type=jax.ShapeDtypeStruct((D, B), jnp.float32),
    mesh=plsc.VectorSubcoreMesh(core_axis_name="c", subcore_axis_name="s"),
    compiler_params=pltpu.CompilerParams(needs_layout_passes=False),
    scratch_types=[
        pltpu.VMEM((FLAT_LEN,), jnp.float32),
        pltpu.VMEM((BPW,), jnp.int32),
        pltpu.VMEM((BPW,), jnp.int32),
        pltpu.VMEM((BPW,), jnp.int32),
        pltpu.VMEM((BPW,), jnp.int32),
        pltpu.VMEM((D, BPW), jnp.float32),
    ],
)(_hero_body)


def kernel(role, race, gend, align, role_table, race_table, gend_table,
           align_table):
    tab2d = jnp.zeros((TAB_ROWS, TAB_COLS), jnp.float32)
    tab2d = lax.dynamic_update_slice(tab2d, role_table, (0, 0))
    tab2d = lax.dynamic_update_slice(tab2d, race_table, (0, ROLE_EMB))
    tab2d = lax.dynamic_update_slice(tab2d, gend_table,
                                     (0, ROLE_EMB + RACE_EMB))
    tab2d = lax.dynamic_update_slice(tab2d, align_table,
                                     (0, ROLE_EMB + RACE_EMB + GEND_EMB))
    tables = tab2d.reshape(-1)
    out_cm = _hero(role.astype(jnp.int32), race.astype(jnp.int32),
                   gend.astype(jnp.int32), align.astype(jnp.int32), tables)
    return out_cm.T
